# Initial kernel scaffold; baseline (speedup 1.0000x reference)
import functools

import jax
import jax.numpy as jnp
from jax import lax
from jax.experimental import pallas as pl
from jax.experimental.pallas import tpu as pltpu
from jax.experimental.pallas import tpu_sc as plsc

N = 10000
E = 320000
D = 128
BN_EPS = 1e-5

NC = 2
NS = 16
NW = NC * NS
EDGES_PER_W = E // NW
CHUNK = 80
CHUNKS_PER_W = EDGES_PER_W // CHUNK
NUM_CHUNKS = E // CHUNK
ROWS_PER_TILE = N // NS


def _sc_aggregate(x, src_c, dst_c, zeros_init):
    mesh = plsc.VectorSubcoreMesh(core_axis_name="c", subcore_axis_name="s",
                                  num_cores=NC, num_subcores=NS)

    @functools.partial(
        pl.kernel,
        out_type=jax.ShapeDtypeStruct((NC, N, D), jnp.float32),
        mesh=mesh,
        scratch_types=[
            pltpu.VMEM_SHARED((N, D), jnp.float32),
            pltpu.VMEM((CHUNKS_PER_W, CHUNK), jnp.int32),
            pltpu.VMEM((CHUNKS_PER_W, CHUNK), jnp.int32),
            pltpu.VMEM((2, CHUNK, D), jnp.float32),
            pltpu.SemaphoreType.DMA,
        ],
    )
    def agg_kernel(x_hbm, src_hbm, dst_hbm, zero_hbm, out_hbm,
                   acc, src_v, dst_v, rows_v, gsem):
        c = lax.axis_index("c")
        s = lax.axis_index("s")
        w = c * NS + s

        r0 = s * ROWS_PER_TILE

        @pl.when(c == 0)
        def _():
            pltpu.sync_copy(x_hbm.at[pl.ds(r0, ROWS_PER_TILE)],
                            acc.at[pl.ds(r0, ROWS_PER_TILE)])

        @pl.when(c != 0)
        def _():
            pltpu.sync_copy(zero_hbm.at[pl.ds(r0, ROWS_PER_TILE)],
                            acc.at[pl.ds(r0, ROWS_PER_TILE)])

        ch0 = w * CHUNKS_PER_W
        pltpu.sync_copy(src_hbm.at[pl.ds(ch0, CHUNKS_PER_W)], src_v)
        pltpu.sync_copy(dst_hbm.at[pl.ds(ch0, CHUNKS_PER_W)], dst_v)
        plsc.subcore_barrier()

        pltpu.async_copy(x_hbm.at[src_v.at[0]], rows_v.at[0], gsem)

        def body(j, _):
            buf = lax.rem(j, 2)
            nbuf = lax.rem(j + 1, 2)

            @pl.when(j + 1 < CHUNKS_PER_W)
            def _():
                pltpu.async_copy(x_hbm.at[src_v.at[j + 1]], rows_v.at[nbuf],
                                 gsem)

            pltpu.make_async_copy(x_hbm.at[src_v.at[j]], rows_v.at[buf],
                                  gsem).wait()
            pltpu.sync_copy(rows_v.at[buf], acc.at[dst_v.at[j]], add=True)
            return 0

        lax.fori_loop(0, CHUNKS_PER_W, body, 0)
        plsc.subcore_barrier()

        pltpu.sync_copy(acc.at[pl.ds(r0, ROWS_PER_TILE)],
                        out_hbm.at[c, pl.ds(r0, ROWS_PER_TILE)])

    return agg_kernel(x, src_c, dst_c, zeros_init)


def _mlp_body(parts_ref, w1_ref, b1_ref, gamma_ref, beta_ref, w2_ref, b2_ref,
              out_ref):
    h = parts_ref[0] + parts_ref[1]
    h1 = jnp.dot(h, w1_ref[...], preferred_element_type=jnp.float32)
    h1 = h1 + b1_ref[...]
    mean = jnp.mean(h1, axis=0, keepdims=True)
    cent = h1 - mean
    var = jnp.mean(cent * cent, axis=0, keepdims=True)
    hn = gamma_ref[...] * cent * lax.rsqrt(var + BN_EPS) + beta_ref[...]
    hr = jnp.maximum(hn, 0.0)
    out = jnp.dot(hr, w2_ref[...], preferred_element_type=jnp.float32)
    out_ref[...] = out + b2_ref[...]


def kernel(x, edge_index, edge_attr, w1, b1, gamma, beta, w2, b2):
    del edge_attr
    src_c = edge_index[0].reshape(NUM_CHUNKS, CHUNK)
    dst_c = edge_index[1].reshape(NUM_CHUNKS, CHUNK)
    zeros_init = jnp.zeros((N, D), dtype=jnp.float32)

    parts = _sc_aggregate(x, src_c, dst_c, zeros_init)

    out = pl.pallas_call(
        _mlp_body,
        out_shape=jax.ShapeDtypeStruct((N, D), jnp.float32),
    )(parts, w1, b1.reshape(1, D), gamma.reshape(1, D), beta.reshape(1, D),
      w2, b2.reshape(1, D))
    return out


# speedup vs baseline: 11.7544x; 11.7544x over previous
import functools

import jax
import jax.numpy as jnp
from jax import lax
from jax.experimental import pallas as pl
from jax.experimental.pallas import tpu as pltpu
from jax.experimental.pallas import tpu_sc as plsc

N = 10000
E = 320000
D = 128
BN_EPS = 1e-5

NC = 2
NS = 16
NW = NC * NS
EDGES_PER_W = E // NW
CHUNK = 80
CHUNKS_PER_W = EDGES_PER_W // CHUNK
NUM_CHUNKS = E // CHUNK
ROWS_PER_TILE = 624
TAIL_ROWS = N - NS * ROWS_PER_TILE


def _sc_aggregate(x, src_c, dst_c, zeros_init):
    mesh = plsc.VectorSubcoreMesh(core_axis_name="c", subcore_axis_name="s",
                                  num_cores=NC, num_subcores=NS)

    @functools.partial(
        pl.kernel,
        out_type=jax.ShapeDtypeStruct((NC, N, D), jnp.float32),
        mesh=mesh,
        scratch_types=[
            pltpu.VMEM_SHARED((N, D), jnp.float32),
            pltpu.VMEM((EDGES_PER_W,), jnp.int32),
            pltpu.VMEM((CHUNKS_PER_W, CHUNK), jnp.int32),
            pltpu.VMEM((2, CHUNK, D), jnp.float32),
            pltpu.SemaphoreType.DMA,
        ],
    )
    def agg_kernel(x_hbm, src_hbm, dst_hbm, zero_hbm, out_hbm,
                   acc, src_v, dst_v, rows_v, gsem):
        c = lax.axis_index("c")
        s = lax.axis_index("s")
        w = c * NS + s

        r0 = s * ROWS_PER_TILE

        @pl.when(c == 0)
        def _():
            pltpu.sync_copy(x_hbm.at[pl.ds(r0, ROWS_PER_TILE)],
                            acc.at[pl.ds(r0, ROWS_PER_TILE)])

            @pl.when(s == 0)
            def _():
                pltpu.sync_copy(x_hbm.at[pl.ds(NS * ROWS_PER_TILE, TAIL_ROWS)],
                                acc.at[pl.ds(NS * ROWS_PER_TILE, TAIL_ROWS)])

        @pl.when(c != 0)
        def _():
            pltpu.sync_copy(zero_hbm.at[pl.ds(r0, ROWS_PER_TILE)],
                            acc.at[pl.ds(r0, ROWS_PER_TILE)])

            @pl.when(s == 0)
            def _():
                pltpu.sync_copy(
                    zero_hbm.at[pl.ds(NS * ROWS_PER_TILE, TAIL_ROWS)],
                    acc.at[pl.ds(NS * ROWS_PER_TILE, TAIL_ROWS)])

        pltpu.sync_copy(src_hbm.at[w], src_v)
        pltpu.sync_copy(dst_hbm.at[w], dst_v)
        plsc.subcore_barrier()

        def src_slice(j):
            return src_v.at[pl.ds(pl.multiple_of(j * CHUNK, 8), CHUNK)]

        pltpu.async_copy(x_hbm.at[src_slice(0)], rows_v.at[0], gsem)

        def body(j, _):
            buf = lax.rem(j, 2)
            nbuf = lax.rem(j + 1, 2)

            @pl.when(j + 1 < CHUNKS_PER_W)
            def _():
                pltpu.async_copy(x_hbm.at[src_slice(j + 1)], rows_v.at[nbuf],
                                 gsem)

            pltpu.make_async_copy(x_hbm.at[src_slice(j)], rows_v.at[buf],
                                  gsem).wait()
            pltpu.sync_copy(rows_v.at[buf], acc.at[dst_v.at[j]], add=True)
            return 0

        lax.fori_loop(0, CHUNKS_PER_W, body, 0)
        plsc.subcore_barrier()

        pltpu.sync_copy(acc.at[pl.ds(r0, ROWS_PER_TILE)],
                        out_hbm.at[c, pl.ds(r0, ROWS_PER_TILE)])

        @pl.when(s == 0)
        def _():
            pltpu.sync_copy(acc.at[pl.ds(NS * ROWS_PER_TILE, TAIL_ROWS)],
                            out_hbm.at[c, pl.ds(NS * ROWS_PER_TILE, TAIL_ROWS)])

    return agg_kernel(x, src_c, dst_c, zeros_init)


def _mlp_body(parts_ref, w1_ref, b1_ref, gamma_ref, beta_ref, w2_ref, b2_ref,
              out_ref):
    h = parts_ref[0] + parts_ref[1]
    h1 = jnp.dot(h, w1_ref[...], preferred_element_type=jnp.float32)
    h1 = h1 + b1_ref[...]
    mean = jnp.mean(h1, axis=0, keepdims=True)
    cent = h1 - mean
    var = jnp.mean(cent * cent, axis=0, keepdims=True)
    hn = gamma_ref[...] * cent * lax.rsqrt(var + BN_EPS) + beta_ref[...]
    hr = jnp.maximum(hn, 0.0)
    out = jnp.dot(hr, w2_ref[...], preferred_element_type=jnp.float32)
    out_ref[...] = out + b2_ref[...]


def kernel(x, edge_index, edge_attr, w1, b1, gamma, beta, w2, b2):
    del edge_attr
    src_c = edge_index[0].reshape(NW, EDGES_PER_W)
    dst_c = edge_index[1].reshape(NW, CHUNKS_PER_W, CHUNK)
    zeros_init = jnp.zeros((N, D), dtype=jnp.float32)

    parts = _sc_aggregate(x, src_c, dst_c, zeros_init)

    out = pl.pallas_call(
        _mlp_body,
        out_shape=jax.ShapeDtypeStruct((N, D), jnp.float32),
    )(parts, w1, b1.reshape(1, D), gamma.reshape(1, D), beta.reshape(1, D),
      w2, b2.reshape(1, D))
    return out
